# baseline (speedup 1.0000x reference)
import functools

import jax
import jax.numpy as jnp
from jax import lax
from jax.experimental import pallas as pl
from jax.experimental.pallas import tpu as pltpu
from jax.experimental.pallas import tpu_sc as plsc

_C = 128


def _build(E, D):
    NC, NS = 2, 16
    NW = NC * NS
    n_chunks = E // _C
    n_i = -(-n_chunks // NW)
    if n_i % 2:
        n_i += 1
    n_pairs = n_i // 2
    mesh = plsc.VectorSubcoreMesh(core_axis_name="c", subcore_axis_name="s")

    @functools.partial(
        pl.kernel,
        mesh=mesh,
        out_type=jax.ShapeDtypeStruct((E, 3 * D), jnp.float32),
        scratch_types=[
            pltpu.VMEM((2, _C), jnp.int32),
            pltpu.VMEM((2, _C), jnp.int32),
            pltpu.VMEM((_C, 3 * D), jnp.float32),
            pltpu.VMEM((_C, 3 * D), jnp.float32),
            pltpu.SemaphoreType.DMA,
            pltpu.SemaphoreType.DMA,
            pltpu.SemaphoreType.DMA,
            pltpu.SemaphoreType.DMA,
            pltpu.SemaphoreType.DMA,
        ],
    )
    def k(edge_hbm, node_hbm, idx_hbm, out_hbm,
          idx0, idx1, rows0, rows1,
          semidx, semg0, semg1, semst0, semst1):
        wid = lax.axis_index("s") * NC + lax.axis_index("c")
        idxb = (idx0, idx1)
        rowb = (rows0, rows1)
        semg = (semg0, semg1)
        semst = (semst0, semst1)

        def chunk_of(i):
            ch = wid + i * NW
            return jnp.where(ch >= n_chunks, ch - n_chunks, ch)

        def load_copies(i, b, sem):
            base = chunk_of(i) * _C
            rows = rowb[b]
            return (
                pltpu.make_async_copy(edge_hbm.at[pl.ds(base, _C)],
                                      rows.at[:, pl.ds(0, D)], sem),
                pltpu.make_async_copy(node_hbm.at[idxb[b].at[0]],
                                      rows.at[:, pl.ds(D, D)], sem),
                pltpu.make_async_copy(node_hbm.at[idxb[b].at[1]],
                                      rows.at[:, pl.ds(2 * D, D)], sem),
            )

        def store_copies(i, b, sem):
            base = chunk_of(i) * _C
            return (
                pltpu.make_async_copy(rowb[b], out_hbm.at[pl.ds(base, _C)],
                                      sem),
            )

        def idx_copy(i, b):
            return pltpu.make_async_copy(idx_hbm.at[chunk_of(i)], idxb[b],
                                         semidx)

        def start(copies):
            for c in copies:
                c.start()

        def drain(copies):
            for c in copies:
                c.wait()

        idx_copy(0, 0).start()
        idx_copy(0, 0).wait()
        start(load_copies(0, 0, semg0))

        def body(p, carry):
            i0 = 2 * p
            i1 = i0 + 1

            idx_copy(i1, 1).start()
            drain(load_copies(i0, 0, semg0))
            start(store_copies(i0, 0, semst0))

            @pl.when(p > 0)
            def _():
                drain(store_copies(i0 - 1, 1, semst1))

            idx_copy(i1, 1).wait()
            start(load_copies(i1, 1, semg1))

            @pl.when(p < n_pairs - 1)
            def _():
                idx_copy(i1 + 1, 0).start()

            drain(load_copies(i1, 1, semg1))
            start(store_copies(i1, 1, semst1))
            drain(store_copies(i0, 0, semst0))

            @pl.when(p < n_pairs - 1)
            def _():
                idx_copy(i1 + 1, 0).wait()
                start(load_copies(i1 + 1, 0, semg0))

            return carry

        lax.fori_loop(0, n_pairs, body, 0)

        drain(store_copies(n_i - 1, 1, semst1))

    return k


def kernel(edge_feat, node_feat, edge_index):
    E, D = edge_feat.shape
    n_chunks = E // _C
    idx = edge_index.astype(jnp.int32)
    comb = idx.reshape(2, n_chunks, _C).transpose(1, 0, 2)
    return _build(E, D)(edge_feat, node_feat, comb)
